# baseline (speedup 1.0000x reference)
import jax
import jax.numpy as jnp
from jax import lax
from jax.experimental import pallas as pl
from jax.experimental.pallas import tpu as pltpu
from jax.experimental.pallas import tpu_sc as plsc

NC = 2
NS = 16
NW = NC * NS
CHUNK = 64
D = 128


def _sc_scatter_x(n_pad, n_chunks, feats, sidx, didx, zrows):
    rows_pt = n_pad // NS

    def body(x_hbm, sidx_hbm, didx_hbm, zrows_hbm, out_hbm,
             acc, si_a, di_a, si_b, di_b, buf_a, buf_b, sem_a, sem_b):
        c = lax.axis_index("c")
        s = lax.axis_index("s")
        wid = c * NS + s
        r0 = s * rows_pt
        pltpu.sync_copy(zrows_hbm, acc.at[pl.ds(r0, rows_pt)])
        plsc.subcore_barrier()

        def pair(q, carry):
            pltpu.sync_copy(sidx_hbm.at[wid, 2 * q], si_a)
            pltpu.sync_copy(didx_hbm.at[wid, 2 * q], di_a)
            pltpu.sync_copy(sidx_hbm.at[wid, 2 * q + 1], si_b)
            pltpu.sync_copy(didx_hbm.at[wid, 2 * q + 1], di_b)
            cp_a = pltpu.async_copy(x_hbm.at[si_a], buf_a, sem_a)
            cp_b = pltpu.async_copy(x_hbm.at[si_b], buf_b, sem_b)
            cp_a.wait()
            pltpu.sync_copy(buf_a, acc.at[di_a], add=True)
            cp_b.wait()
            pltpu.sync_copy(buf_b, acc.at[di_b], add=True)
            return carry

        lax.fori_loop(0, n_chunks // 2, pair, 0)
        plsc.subcore_barrier()
        pltpu.sync_copy(acc.at[pl.ds(r0, rows_pt)],
                        out_hbm.at[c, pl.ds(r0, rows_pt)])

    f = pl.kernel(
        body,
        out_type=jax.ShapeDtypeStruct((NC, n_pad, D), jnp.float32),
        mesh=plsc.VectorSubcoreMesh(core_axis_name="c", subcore_axis_name="s"),
        scratch_types=[
            pltpu.VMEM_SHARED((n_pad, D), jnp.float32),
            pltpu.VMEM((CHUNK,), jnp.int32),
            pltpu.VMEM((CHUNK,), jnp.int32),
            pltpu.VMEM((CHUNK,), jnp.int32),
            pltpu.VMEM((CHUNK,), jnp.int32),
            pltpu.VMEM((CHUNK, D), jnp.float32),
            pltpu.VMEM((CHUNK, D), jnp.float32),
            pltpu.SemaphoreType.DMA,
            pltpu.SemaphoreType.DMA,
        ],
    )
    return f(feats, sidx, didx, zrows)


def _sc_scatter_aux(n_pad, n_chunks, attr4, didx, zrows, tmpl):
    rows_pt = n_pad // NS

    def body(attr_hbm, didx_hbm, zrows_hbm, tmpl_hbm, out_hbm,
             acc, didx_v, attr_v, aux_v):
        c = lax.axis_index("c")
        s = lax.axis_index("s")
        wid = c * NS + s
        r0 = s * rows_pt
        pltpu.sync_copy(zrows_hbm, acc.at[pl.ds(r0, rows_pt)])
        pltpu.sync_copy(tmpl_hbm, aux_v)
        plsc.subcore_barrier()

        def chunk(j, carry):
            pltpu.sync_copy(didx_hbm.at[wid, j], didx_v)
            pltpu.sync_copy(attr_hbm.at[wid, j], attr_v)
            for r in range(8):
                for k in range(8):
                    aux_v[r * 8 + k, 0:16] = attr_v[r, k * 16:(k + 1) * 16]
            pltpu.sync_copy(aux_v, acc.at[didx_v], add=True)
            return carry

        lax.fori_loop(0, n_chunks, chunk, 0)
        plsc.subcore_barrier()
        pltpu.sync_copy(acc.at[pl.ds(r0, rows_pt)],
                        out_hbm.at[c, pl.ds(r0, rows_pt)])

    f = pl.kernel(
        body,
        out_type=jax.ShapeDtypeStruct((NC, n_pad, D), jnp.float32),
        mesh=plsc.VectorSubcoreMesh(core_axis_name="c", subcore_axis_name="s"),
        scratch_types=[
            pltpu.VMEM_SHARED((n_pad, D), jnp.float32),
            pltpu.VMEM((CHUNK,), jnp.int32),
            pltpu.VMEM((8, D), jnp.float32),
            pltpu.VMEM((CHUNK, D), jnp.float32),
        ],
    )
    return f(attr4, didx, zrows, tmpl)


def _tc_dense(accx, accaux, xin, We, be, Wl, bl, Wr, act):
    n_pad, d = xin.shape
    br = 1024
    grid = (n_pad // br,)

    def body(ax0, ax1, au0, au1, xr, we, be_r, wl, bl_r, wr, out):
        aux = au0[...] + au1[...]
        attr = aux[:, 0:16]
        deg = aux[:, 16:17]
        aggr = (ax0[...] + ax1[...]
                + jnp.dot(attr, we[...], preferred_element_type=jnp.float32)
                + deg * be_r[...])
        aggr = aggr / jnp.maximum(deg, 1.0)
        val = (jnp.dot(aggr, wl[...], preferred_element_type=jnp.float32)
               + bl_r[...]
               + jnp.dot(xr[...], wr[...], preferred_element_type=jnp.float32))
        out[...] = act(val)

    rd = pl.BlockSpec((br, d), lambda i: (i, 0))

    def full(a):
        return pl.BlockSpec(a.shape, lambda i: (0,) * a.ndim)

    be2 = be.reshape(1, -1)
    bl2 = bl.reshape(1, -1)
    return pl.pallas_call(
        body,
        grid=grid,
        in_specs=[rd, rd, rd, rd, rd,
                  full(We), full(be2), full(Wl), full(bl2), full(Wr)],
        out_specs=rd,
        out_shape=jax.ShapeDtypeStruct((n_pad, d), jnp.float32),
    )(accx[0], accx[1], accaux[0], accaux[1], xin,
      We, be2, Wl, bl2, Wr)


def kernel(x, edge_index1, edge_attr1, edge_index2, edge_attr2,
           We1, be1, Wl1, bl1, Wr1, We2, be2, Wl2, bl2, Wr2):
    n, d = x.shape
    e = edge_index1.shape[1]
    da = edge_attr1.shape[1]
    n_pad = -(-n // 1024) * 1024
    e_pad = -(-e // (2 * NW * CHUNK)) * (2 * NW * CHUNK)
    n_chunks = e_pad // (NW * CHUNK)
    pad = e_pad - e

    def prep_edges(ei, ea):
        src = jnp.pad(ei[0].astype(jnp.int32), (0, pad))
        src = src.reshape(NW, n_chunks, CHUNK)
        dst = jnp.pad(ei[1].astype(jnp.int32), (0, pad),
                      constant_values=n)
        dst = dst.reshape(NW, n_chunks, CHUNK)
        attr4 = jnp.pad(ea, ((0, pad), (0, 0))).reshape(NW, n_chunks, 8, 128)
        return src, dst, attr4

    s1, d1, a1 = prep_edges(edge_index1, edge_attr1)
    s2, d2, a2 = prep_edges(edge_index2, edge_attr2)
    x_pad = jnp.pad(x, ((0, n_pad - n), (0, 0)))
    zrows = jnp.zeros((n_pad // NS, D), jnp.float32)
    tmpl = jnp.tile(jnp.concatenate([jnp.zeros((1, da), jnp.float32),
                                     jnp.ones((1, 1), jnp.float32),
                                     jnp.zeros((1, D - da - 1), jnp.float32)],
                                    axis=1), (CHUNK, 1))

    ax = _sc_scatter_x(n_pad, n_chunks, x_pad, s1, d1, zrows)
    au = _sc_scatter_aux(n_pad, n_chunks, a1, d1, zrows, tmpl)
    h = _tc_dense(ax, au, x_pad, We1, be1, Wl1, bl1, Wr1, jax.nn.relu)
    ax2 = _sc_scatter_x(n_pad, n_chunks, h, s2, d2, zrows)
    au2 = _sc_scatter_aux(n_pad, n_chunks, a2, d2, zrows, tmpl)
    out = _tc_dense(ax2, au2, h, We2, be2, Wl2, bl2, Wr2, jax.nn.sigmoid)
    return out[:n]


# speedup vs baseline: 1.1942x; 1.1942x over previous
import jax
import jax.numpy as jnp
from jax import lax
from jax.experimental import pallas as pl
from jax.experimental.pallas import tpu as pltpu
from jax.experimental.pallas import tpu_sc as plsc

NC = 2
NS = 16
NW = NC * NS
CHUNK = 64
D = 128


def _sc_scatter_x(n_pad, n_chunks, feats, sidx, didx, zrows):
    rows_pt = n_pad // NS

    def body(x_hbm, sidx_hbm, didx_hbm, zrows_hbm, out_hbm,
             acc, si_all, di_all, rows0, sem0):
        c = lax.axis_index("c")
        s = lax.axis_index("s")
        wid = c * NS + s
        r0 = s * rows_pt
        pltpu.sync_copy(zrows_hbm, acc.at[pl.ds(r0, rows_pt)])
        pltpu.sync_copy(sidx_hbm.at[wid], si_all)
        pltpu.sync_copy(didx_hbm.at[wid], di_all)
        plsc.subcore_barrier()

        def chunk(j, carry):
            pltpu.async_copy(x_hbm.at[si_all.at[j]], rows0, sem0).wait()
            pltpu.sync_copy(rows0, acc.at[di_all.at[j]], add=True)
            return carry

        lax.fori_loop(0, n_chunks, chunk, 0)
        plsc.subcore_barrier()
        pltpu.sync_copy(acc.at[pl.ds(r0, rows_pt)],
                        out_hbm.at[c, pl.ds(r0, rows_pt)])

    f = pl.kernel(
        body,
        out_type=jax.ShapeDtypeStruct((NC, n_pad, D), jnp.float32),
        mesh=plsc.VectorSubcoreMesh(core_axis_name="c", subcore_axis_name="s"),
        scratch_types=[
            pltpu.VMEM_SHARED((n_pad, D), jnp.float32),
            pltpu.VMEM((n_chunks, CHUNK), jnp.int32),
            pltpu.VMEM((n_chunks, CHUNK), jnp.int32),
            pltpu.VMEM((CHUNK, D), jnp.float32),
            pltpu.SemaphoreType.DMA,
        ],
    )
    return f(feats, sidx, didx, zrows)


def _sc_scatter_aux(n_pad, n_chunks, attr4, didx, zrows, tmpl):
    rows_pt = n_pad // NS

    def body(attr_hbm, didx_hbm, zrows_hbm, tmpl_hbm, out_hbm,
             acc, di_all, attr_v, aux_v):
        c = lax.axis_index("c")
        s = lax.axis_index("s")
        wid = c * NS + s
        r0 = s * rows_pt
        pltpu.sync_copy(zrows_hbm, acc.at[pl.ds(r0, rows_pt)])
        pltpu.sync_copy(didx_hbm.at[wid], di_all)
        pltpu.sync_copy(tmpl_hbm, aux_v)
        plsc.subcore_barrier()

        def chunk(j, carry):
            pltpu.sync_copy(attr_hbm.at[wid, j], attr_v)
            for r in range(8):
                for k in range(8):
                    aux_v[r * 8 + k, 0:16] = attr_v[r, k * 16:(k + 1) * 16]
            pltpu.sync_copy(aux_v, acc.at[di_all.at[j]], add=True)
            return carry

        lax.fori_loop(0, n_chunks, chunk, 0)
        plsc.subcore_barrier()
        pltpu.sync_copy(acc.at[pl.ds(r0, rows_pt)],
                        out_hbm.at[c, pl.ds(r0, rows_pt)])

    f = pl.kernel(
        body,
        out_type=jax.ShapeDtypeStruct((NC, n_pad, D), jnp.float32),
        mesh=plsc.VectorSubcoreMesh(core_axis_name="c", subcore_axis_name="s"),
        scratch_types=[
            pltpu.VMEM_SHARED((n_pad, D), jnp.float32),
            pltpu.VMEM((n_chunks, CHUNK), jnp.int32),
            pltpu.VMEM((8, D), jnp.float32),
            pltpu.VMEM((CHUNK, D), jnp.float32),
        ],
    )
    return f(attr4, didx, zrows, tmpl)


def _tc_dense(accx, accaux, xin, We, be, Wl, bl, Wr, act):
    n_pad, d = xin.shape
    br = 1024
    grid = (n_pad // br,)

    def body(ax0, ax1, au0, au1, xr, we, be_r, wl, bl_r, wr, out):
        aux = au0[...] + au1[...]
        attr = aux[:, 0:16]
        deg = aux[:, 16:17]
        aggr = (ax0[...] + ax1[...]
                + jnp.dot(attr, we[...], preferred_element_type=jnp.float32)
                + deg * be_r[...])
        aggr = aggr / jnp.maximum(deg, 1.0)
        val = (jnp.dot(aggr, wl[...], preferred_element_type=jnp.float32)
               + bl_r[...]
               + jnp.dot(xr[...], wr[...], preferred_element_type=jnp.float32))
        out[...] = act(val)

    rd = pl.BlockSpec((br, d), lambda i: (i, 0))

    def full(a):
        return pl.BlockSpec(a.shape, lambda i: (0,) * a.ndim)

    be2 = be.reshape(1, -1)
    bl2 = bl.reshape(1, -1)
    return pl.pallas_call(
        body,
        grid=grid,
        in_specs=[rd, rd, rd, rd, rd,
                  full(We), full(be2), full(Wl), full(bl2), full(Wr)],
        out_specs=rd,
        out_shape=jax.ShapeDtypeStruct((n_pad, d), jnp.float32),
    )(accx[0], accx[1], accaux[0], accaux[1], xin,
      We, be2, Wl, bl2, Wr)


def kernel(x, edge_index1, edge_attr1, edge_index2, edge_attr2,
           We1, be1, Wl1, bl1, Wr1, We2, be2, Wl2, bl2, Wr2):
    n, d = x.shape
    e = edge_index1.shape[1]
    da = edge_attr1.shape[1]
    n_pad = -(-n // 1024) * 1024
    e_pad = -(-e // (2 * NW * CHUNK)) * (2 * NW * CHUNK)
    n_chunks = e_pad // (NW * CHUNK)
    pad = e_pad - e

    def prep_edges(ei, ea):
        src = jnp.pad(ei[0].astype(jnp.int32), (0, pad))
        src = src.reshape(NW, n_chunks, CHUNK)
        dst = jnp.pad(ei[1].astype(jnp.int32), (0, pad),
                      constant_values=n)
        dst = dst.reshape(NW, n_chunks, CHUNK)
        attr4 = jnp.pad(ea, ((0, pad), (0, 0))).reshape(NW, n_chunks, 8, 128)
        return src, dst, attr4

    s1, d1, a1 = prep_edges(edge_index1, edge_attr1)
    s2, d2, a2 = prep_edges(edge_index2, edge_attr2)
    x_pad = jnp.pad(x, ((0, n_pad - n), (0, 0)))
    zrows = jnp.zeros((n_pad // NS, D), jnp.float32)
    tmpl = jnp.tile(jnp.concatenate([jnp.zeros((1, da), jnp.float32),
                                     jnp.ones((1, 1), jnp.float32),
                                     jnp.zeros((1, D - da - 1), jnp.float32)],
                                    axis=1), (CHUNK, 1))

    ax = _sc_scatter_x(n_pad, n_chunks, x_pad, s1, d1, zrows)
    au = _sc_scatter_aux(n_pad, n_chunks, a1, d1, zrows, tmpl)
    h = _tc_dense(ax, au, x_pad, We1, be1, Wl1, bl1, Wr1, jax.nn.relu)
    ax2 = _sc_scatter_x(n_pad, n_chunks, h, s2, d2, zrows)
    au2 = _sc_scatter_aux(n_pad, n_chunks, a2, d2, zrows, tmpl)
    out = _tc_dense(ax2, au2, h, We2, be2, Wl2, bl2, Wr2, jax.nn.sigmoid)
    return out[:n]


# speedup vs baseline: 1.2402x; 1.0386x over previous
import jax
import jax.numpy as jnp
from jax import lax
from jax.experimental import pallas as pl
from jax.experimental.pallas import tpu as pltpu
from jax.experimental.pallas import tpu_sc as plsc

NC = 2
NS = 16
NW = NC * NS
CHUNK = 64
D = 128


def _sc_scatter_x(n_pad, n_chunks, feats, pidx, zrows):
    rows_pt = n_pad // NS

    def body(x_hbm, pidx_hbm, zrows_hbm, out_hbm,
             acc, pk_all, si0, di0, si1, di1, rows0, rows1, sem0, sem1):
        c = lax.axis_index("c")
        s = lax.axis_index("s")
        wid = c * NS + s
        r0 = s * rows_pt
        pltpu.sync_copy(zrows_hbm, acc.at[pl.ds(r0, rows_pt)])
        pltpu.sync_copy(pidx_hbm.at[wid], pk_all)
        plsc.subcore_barrier()

        def pair(p, carry):
            j = 2 * p
            for k in range(CHUNK // 16):
                sl = slice(16 * k, 16 * (k + 1))
                v0 = pk_all[j, sl]
                si0[sl] = lax.shift_right_logical(v0, 16)
                di0[sl] = lax.bitwise_and(v0, 0xFFFF)
                v1 = pk_all[j + 1, sl]
                si1[sl] = lax.shift_right_logical(v1, 16)
                di1[sl] = lax.bitwise_and(v1, 0xFFFF)
            c0 = pltpu.async_copy(x_hbm.at[si0], rows0, sem0)
            c1 = pltpu.async_copy(x_hbm.at[si1], rows1, sem1)
            c0.wait()
            pltpu.sync_copy(rows0, acc.at[di0], add=True)
            c1.wait()
            pltpu.sync_copy(rows1, acc.at[di1], add=True)
            return carry

        lax.fori_loop(0, n_chunks // 2, pair, 0)
        plsc.subcore_barrier()
        pltpu.sync_copy(acc.at[pl.ds(r0, rows_pt)],
                        out_hbm.at[c, pl.ds(r0, rows_pt)])

    f = pl.kernel(
        body,
        out_type=jax.ShapeDtypeStruct((NC, n_pad, D), jnp.float32),
        mesh=plsc.VectorSubcoreMesh(core_axis_name="c", subcore_axis_name="s"),
        scratch_types=[
            pltpu.VMEM_SHARED((n_pad, D), jnp.float32),
            pltpu.VMEM((n_chunks, CHUNK), jnp.int32),
            pltpu.VMEM((CHUNK,), jnp.int32),
            pltpu.VMEM((CHUNK,), jnp.int32),
            pltpu.VMEM((CHUNK,), jnp.int32),
            pltpu.VMEM((CHUNK,), jnp.int32),
            pltpu.VMEM((CHUNK, D), jnp.float32),
            pltpu.VMEM((CHUNK, D), jnp.float32),
            pltpu.SemaphoreType.DMA,
            pltpu.SemaphoreType.DMA,
        ],
    )
    return f(feats, pidx, zrows)


def _sc_scatter_aux(n_pad, n_chunks, attr4, didx, zrows, tmpl):
    rows_pt = n_pad // NS

    def body(attr_hbm, didx_hbm, zrows_hbm, tmpl_hbm, out_hbm,
             acc, di_all, attr_v, aux_v):
        c = lax.axis_index("c")
        s = lax.axis_index("s")
        wid = c * NS + s
        r0 = s * rows_pt
        pltpu.sync_copy(zrows_hbm, acc.at[pl.ds(r0, rows_pt)])
        pltpu.sync_copy(didx_hbm.at[wid], di_all)
        pltpu.sync_copy(tmpl_hbm, aux_v)
        plsc.subcore_barrier()

        def chunk(j, carry):
            pltpu.sync_copy(attr_hbm.at[wid, j], attr_v)
            for r in range(8):
                for k in range(8):
                    aux_v[r * 8 + k, 0:16] = attr_v[r, k * 16:(k + 1) * 16]
            pltpu.sync_copy(aux_v, acc.at[di_all.at[j]], add=True)
            return carry

        lax.fori_loop(0, n_chunks, chunk, 0)
        plsc.subcore_barrier()
        pltpu.sync_copy(acc.at[pl.ds(r0, rows_pt)],
                        out_hbm.at[c, pl.ds(r0, rows_pt)])

    f = pl.kernel(
        body,
        out_type=jax.ShapeDtypeStruct((NC, n_pad, D), jnp.float32),
        mesh=plsc.VectorSubcoreMesh(core_axis_name="c", subcore_axis_name="s"),
        scratch_types=[
            pltpu.VMEM_SHARED((n_pad, D), jnp.float32),
            pltpu.VMEM((n_chunks, CHUNK), jnp.int32),
            pltpu.VMEM((8, D), jnp.float32),
            pltpu.VMEM((CHUNK, D), jnp.float32),
        ],
    )
    return f(attr4, didx, zrows, tmpl)


def _tc_dense(accx, accaux, xin, We, be, Wl, bl, Wr, act):
    n_pad, d = xin.shape
    br = 1024
    grid = (n_pad // br,)

    def body(ax0, ax1, au0, au1, xr, we, be_r, wl, bl_r, wr, out):
        aux = au0[...] + au1[...]
        attr = aux[:, 0:16]
        deg = aux[:, 16:17]
        aggr = (ax0[...] + ax1[...]
                + jnp.dot(attr, we[...], preferred_element_type=jnp.float32)
                + deg * be_r[...])
        aggr = aggr / jnp.maximum(deg, 1.0)
        val = (jnp.dot(aggr, wl[...], preferred_element_type=jnp.float32)
               + bl_r[...]
               + jnp.dot(xr[...], wr[...], preferred_element_type=jnp.float32))
        out[...] = act(val)

    rd = pl.BlockSpec((br, d), lambda i: (i, 0))

    def full(a):
        return pl.BlockSpec(a.shape, lambda i: (0,) * a.ndim)

    be2 = be.reshape(1, -1)
    bl2 = bl.reshape(1, -1)
    return pl.pallas_call(
        body,
        grid=grid,
        in_specs=[rd, rd, rd, rd, rd,
                  full(We), full(be2), full(Wl), full(bl2), full(Wr)],
        out_specs=rd,
        out_shape=jax.ShapeDtypeStruct((n_pad, d), jnp.float32),
    )(accx[0], accx[1], accaux[0], accaux[1], xin,
      We, be2, Wl, bl2, Wr)


def kernel(x, edge_index1, edge_attr1, edge_index2, edge_attr2,
           We1, be1, Wl1, bl1, Wr1, We2, be2, Wl2, bl2, Wr2):
    n, d = x.shape
    e = edge_index1.shape[1]
    da = edge_attr1.shape[1]
    n_pad = -(-n // 1024) * 1024
    e_pad = -(-e // (2 * NW * CHUNK)) * (2 * NW * CHUNK)
    n_chunks = e_pad // (NW * CHUNK)
    pad = e_pad - e

    def prep_edges(ei, ea):
        src = jnp.pad(ei[0].astype(jnp.int32), (0, pad))
        dst = jnp.pad(ei[1].astype(jnp.int32), (0, pad),
                      constant_values=n)
        pk = ((src << 16) | dst).reshape(NW, n_chunks, CHUNK)
        dst = dst.reshape(NW, n_chunks, CHUNK)
        attr4 = jnp.pad(ea, ((0, pad), (0, 0))).reshape(NW, n_chunks, 8, 128)
        return pk, dst, attr4

    p1, d1, a1 = prep_edges(edge_index1, edge_attr1)
    p2, d2, a2 = prep_edges(edge_index2, edge_attr2)
    x_pad = jnp.pad(x, ((0, n_pad - n), (0, 0)))
    zrows = jnp.zeros((n_pad // NS, D), jnp.float32)
    tmpl = jnp.tile(jnp.concatenate([jnp.zeros((1, da), jnp.float32),
                                     jnp.ones((1, 1), jnp.float32),
                                     jnp.zeros((1, D - da - 1), jnp.float32)],
                                    axis=1), (CHUNK, 1))

    ax = _sc_scatter_x(n_pad, n_chunks, x_pad, p1, zrows)
    au = _sc_scatter_aux(n_pad, n_chunks, a1, d1, zrows, tmpl)
    h = _tc_dense(ax, au, x_pad, We1, be1, Wl1, bl1, Wr1, jax.nn.relu)
    ax2 = _sc_scatter_x(n_pad, n_chunks, h, p2, zrows)
    au2 = _sc_scatter_aux(n_pad, n_chunks, a2, d2, zrows, tmpl)
    out = _tc_dense(ax2, au2, h, We2, be2, Wl2, bl2, Wr2, jax.nn.sigmoid)
    return out[:n]
